# baseline (speedup 1.0000x reference)
import functools

import numpy as np
import jax
import jax.numpy as jnp
from jax.experimental import pallas as pl
from jax.experimental.pallas import tpu as pltpu

_VMEM_LIMIT = 96 * 1024 * 1024


@functools.lru_cache(maxsize=None)
def _pool_matrix_full(w, h, pw, ph, fw, fh):
    H0 = h * fh
    M = pw * ph
    P = np.zeros((w * fw * H0, fw * fh * M), dtype=np.float32)
    for f1 in range(fw):
        for f2 in range(fh):
            for i in range(pw):
                ws, we = (i * w) // pw, -(-((i + 1) * w) // pw)
                for j in range(ph):
                    hs, he = (j * h) // ph, -(-((j + 1) * h) // ph)
                    cnt = float((we - ws) * (he - hs))
                    for iw in range(ws, we):
                        for ih in range(hs, he):
                            p = (f1 * w + iw) * H0 + f2 * h + ih
                            P[p, (f1 * fh + f2) * M + i * ph + j] = 1.0 / cnt
    return P


def _fused_kernel(ab_ref, wcv_ref, bcv_ref, wo_ref, bo_ref, pool_ref,
                  x_ref, o_ref, *, heads, head_dim, M, w, h, H0, fold_w,
                  fold_h):
    alpha = ab_ref[0]
    beta = ab_ref[1]
    H, hd = heads, head_dim
    Cd = H * hd
    f32 = jnp.float32
    bf16 = jnp.bfloat16
    Ph = w * H0
    FMh = fold_h * M

    wcen = wcv_ref[:Cd]
    bcen = bcv_ref[:Cd]
    wval_b = wcv_ref[Cd:].astype(bf16)
    bval = bcv_ref[Cd:]
    wo_b = wo_ref[...].astype(bf16)

    rows = jax.lax.broadcasted_iota(jnp.int32, (FMh, Ph), 0)
    pix = jax.lax.broadcasted_iota(jnp.int32, (FMh, Ph), 1)
    own = (rows // M) == (pix % H0) // h
    brow = jax.lax.broadcasted_iota(jnp.int32, (Cd, H * FMh), 0)
    bcol = jax.lax.broadcasted_iota(jnp.int32, (Cd, H * FMh), 1)
    bmask = (brow // hd) == (bcol // FMh)

    for i in range(x_ref.shape[0]):
        for j in range(fold_w):
            x = x_ref[i][:, j * Ph:(j + 1) * Ph]
            x_b = x.astype(bf16)
            pool = pool_ref[j * Ph:(j + 1) * Ph,
                            j * FMh:(j + 1) * FMh]
            cen = (jnp.dot(wcen, x, preferred_element_type=f32)
                   + bcen)
            val = ((jnp.dot(wval_b, x_b, preferred_element_type=f32)
                    + bval).astype(bf16))

            ccen = jnp.dot(cen, pool, preferred_element_type=f32)
            cval = jnp.dot(val, pool.astype(bf16),
                           preferred_element_type=f32)

            hard_l, hdisp_l = [], []
            for e in range(H):
                ce = cen[e * hd:(e + 1) * hd]
                cc = ccen[e * hd:(e + 1) * hd]
                xn = ce * jax.lax.rsqrt(jnp.maximum(
                    jnp.sum(ce * ce, axis=0, keepdims=True), 1e-24))
                cn = cc * jax.lax.rsqrt(jnp.maximum(
                    jnp.sum(cc * cc, axis=0, keepdims=True), 1e-24))
                cos = jax.lax.dot_general(
                    cn, xn, (((0,), (0,)), ((), ())),
                    preferred_element_type=f32)
                masked = jnp.where(own, alpha * cos, -3e38)
                smax = jnp.max(masked, axis=0, keepdims=True)
                idx = jnp.min(jnp.where(masked == smax, rows, FMh),
                              axis=0, keepdims=True)
                simw = jax.nn.sigmoid(beta + smax)
                hard_e = jnp.where(rows == idx, simw, 0.0)
                denom = jnp.sum(hard_e, axis=1, keepdims=True) + 1.0
                hard_l.append(hard_e.astype(bf16))
                hdisp_l.append((hard_e / denom).astype(bf16))
            hard = jnp.concatenate(hard_l, axis=0)
            hdisp = jnp.concatenate(hdisp_l, axis=0)

            cu = jax.lax.dot_general(val, hard, (((1,), (1,)), ((), ())),
                                     preferred_element_type=f32)
            cvt = jnp.concatenate([cval] * H, axis=1)
            g = jnp.where(bmask, cu + cvt, 0.0).astype(bf16)
            cu2 = jnp.dot(wo_b, g, preferred_element_type=f32).astype(bf16)
            out = (jnp.dot(cu2, hdisp, preferred_element_type=f32)
                   + bo_ref[...])
            o_ref[i, :, j * Ph:(j + 1) * Ph] = out.astype(o_ref.dtype)


def _cluster_attention(x, wp, bp, wc, bc, wo, bo, sim_alpha, sim_beta, *,
                       heads, head_dim, fold_w, fold_h, proposal_w,
                       proposal_h, block_b=2):
    B, dim, W0, H0 = x.shape
    out_dim = wo.shape[0]
    Cd = heads * head_dim
    fw, fh = (fold_w, fold_h) if (fold_w > 1 and fold_h > 1) else (1, 1)
    w, h = W0 // fw, H0 // fh
    M = proposal_w * proposal_h
    P = W0 * H0
    FM = fw * fh * M
    BB = block_b if B % block_b == 0 else 1

    w_cv = jnp.concatenate([wc, wp], axis=0)
    b_cv = jnp.concatenate([bc, bp], axis=0).reshape(2 * Cd, 1)
    ab = jnp.stack([jnp.asarray(sim_alpha, jnp.float32),
                    jnp.asarray(sim_beta, jnp.float32)])
    pool = jnp.asarray(_pool_matrix_full(w, h, proposal_w, proposal_h, fw, fh))
    x_flat = x.reshape(B, dim, P)

    out = pl.pallas_call(
        functools.partial(_fused_kernel, heads=heads, head_dim=head_dim,
                          M=M, w=w, h=h, H0=H0, fold_w=fw, fold_h=fh),
        out_shape=jax.ShapeDtypeStruct((B, out_dim, P), jnp.float32),
        grid=(B // BB,),
        in_specs=[
            pl.BlockSpec(memory_space=pltpu.MemorySpace.SMEM),
            pl.BlockSpec((2 * Cd, dim), lambda g: (0, 0)),
            pl.BlockSpec((2 * Cd, 1), lambda g: (0, 0)),
            pl.BlockSpec((out_dim, Cd), lambda g: (0, 0)),
            pl.BlockSpec((out_dim, 1), lambda g: (0, 0)),
            pl.BlockSpec((P, FM), lambda g: (0, 0)),
            pl.BlockSpec((BB, dim, P), lambda g: (g, 0, 0)),
        ],
        out_specs=pl.BlockSpec((BB, out_dim, P), lambda g: (g, 0, 0)),
        compiler_params=pltpu.CompilerParams(
            dimension_semantics=("parallel",),
            vmem_limit_bytes=_VMEM_LIMIT),
    )(ab, w_cv, b_cv, wo, bo.reshape(out_dim, 1), pool, x_flat)
    return out.reshape(B, out_dim, W0, H0)


@jax.jit
def kernel(x, wp, bp, wc, bc, wo, bo, sim_alpha, sim_beta):
    return _cluster_attention(x, wp, bp, wc, bc, wo, bo, sim_alpha, sim_beta,
                              heads=4, head_dim=32, fold_w=2, fold_h=2,
                              proposal_w=2, proposal_h=2)


# speedup vs baseline: 1.0479x; 1.0479x over previous
import functools

import numpy as np
import jax
import jax.numpy as jnp
from jax.experimental import pallas as pl
from jax.experimental.pallas import tpu as pltpu

_VMEM_LIMIT = 96 * 1024 * 1024


@functools.lru_cache(maxsize=None)
def _pool_matrix_full(w, h, pw, ph, fw, fh):
    H0 = h * fh
    M = pw * ph
    P = np.zeros((w * fw * H0, fw * fh * M), dtype=np.float32)
    for f1 in range(fw):
        for f2 in range(fh):
            for i in range(pw):
                ws, we = (i * w) // pw, -(-((i + 1) * w) // pw)
                for j in range(ph):
                    hs, he = (j * h) // ph, -(-((j + 1) * h) // ph)
                    cnt = float((we - ws) * (he - hs))
                    for iw in range(ws, we):
                        for ih in range(hs, he):
                            p = (f1 * w + iw) * H0 + f2 * h + ih
                            P[p, (f1 * fh + f2) * M + i * ph + j] = 1.0 / cnt
    return P


def _fused_kernel(ab_ref, wcv_ref, bcv_ref, wo_ref, bo_ref, pool_ref,
                  x_ref, o_ref, *, heads, head_dim, M, w, h, H0, fold_h):
    alpha = ab_ref[0]
    beta = ab_ref[1]
    H, hd = heads, head_dim
    Cd = H * hd
    pool = pool_ref[...]
    P, FM = pool.shape
    f32 = jnp.float32
    bf16 = jnp.bfloat16
    wval_b = wcv_ref[Cd:].astype(bf16)
    pool_b = pool.astype(bf16)
    wo_b = wo_ref[...].astype(bf16)

    rows = jax.lax.broadcasted_iota(jnp.int32, (FM, P), 0)
    pix = jax.lax.broadcasted_iota(jnp.int32, (FM, P), 1)
    own = (rows // M) == (pix // (w * H0)) * fold_h + (pix % H0) // h
    brow = jax.lax.broadcasted_iota(jnp.int32, (Cd, H * FM), 0)
    bcol = jax.lax.broadcasted_iota(jnp.int32, (Cd, H * FM), 1)
    bmask = (brow // hd) == (bcol // FM)

    for i in range(x_ref.shape[0]):
        x = x_ref[i]
        x_b = x.astype(bf16)
        cen = (jnp.dot(wcv_ref[:Cd], x, preferred_element_type=f32)
               + bcv_ref[:Cd])

        ccen = jnp.dot(cen, pool, preferred_element_type=f32)
        xpool = jnp.dot(x_b, pool_b,
                        preferred_element_type=f32).astype(bf16)
        cval = (jnp.dot(wval_b, xpool, preferred_element_type=f32)
                + bcv_ref[Cd:])

        hard_l, hdisp_l, denom_l = [], [], []
        for e in range(H):
            ce = cen[e * hd:(e + 1) * hd]
            cc = ccen[e * hd:(e + 1) * hd]
            xn = ce * jax.lax.rsqrt(jnp.maximum(
                jnp.sum(ce * ce, axis=0, keepdims=True), 1e-24))
            cn = cc * jax.lax.rsqrt(jnp.maximum(
                jnp.sum(cc * cc, axis=0, keepdims=True), 1e-24))
            cos = jax.lax.dot_general(cn, xn, (((0,), (0,)), ((), ())),
                                      preferred_element_type=f32)
            masked = jnp.where(own, alpha * cos, -3e38)
            smax = jnp.max(masked, axis=0, keepdims=True)
            idx = jnp.min(jnp.where(masked == smax, rows, FM), axis=0,
                          keepdims=True)
            simw = jax.nn.sigmoid(beta + smax)
            hard_e = jnp.where(rows == idx, simw, 0.0)
            denom = jnp.sum(hard_e, axis=1, keepdims=True) + 1.0
            denom_l.append(denom)
            hard_l.append(hard_e.astype(bf16))
            hdisp_l.append((hard_e / denom).astype(bf16))
        hard = jnp.concatenate(hard_l, axis=0)
        hdisp = jnp.concatenate(hdisp_l, axis=0)

        t1 = jax.lax.dot_general(x_b, hard, (((1,), (1,)), ((), ())),
                                 preferred_element_type=f32)
        hsums = (jnp.concatenate(denom_l, axis=0) - 1.0).reshape(1, H * FM)
        cu = (jnp.dot(wval_b, t1.astype(bf16), preferred_element_type=f32)
              + bcv_ref[Cd:] * hsums)
        cvt = jnp.concatenate([cval] * H, axis=1)
        g = jnp.where(bmask, cu + cvt, 0.0).astype(bf16)
        cu2 = jnp.dot(wo_b, g, preferred_element_type=f32).astype(bf16)
        out = (jnp.dot(cu2, hdisp, preferred_element_type=f32)
               + bo_ref[...])
        o_ref[i] = out.astype(o_ref.dtype)


def _cluster_attention(x, wp, bp, wc, bc, wo, bo, sim_alpha, sim_beta, *,
                       heads, head_dim, fold_w, fold_h, proposal_w,
                       proposal_h, block_b=2):
    B, dim, W0, H0 = x.shape
    out_dim = wo.shape[0]
    Cd = heads * head_dim
    fw, fh = (fold_w, fold_h) if (fold_w > 1 and fold_h > 1) else (1, 1)
    w, h = W0 // fw, H0 // fh
    M = proposal_w * proposal_h
    P = W0 * H0
    FM = fw * fh * M
    BB = block_b if B % block_b == 0 else 1

    w_cv = jnp.concatenate([wc, wp], axis=0)
    b_cv = jnp.concatenate([bc, bp], axis=0).reshape(2 * Cd, 1)
    ab = jnp.stack([jnp.asarray(sim_alpha, jnp.float32),
                    jnp.asarray(sim_beta, jnp.float32)])
    pool = jnp.asarray(_pool_matrix_full(w, h, proposal_w, proposal_h, fw, fh))
    x_flat = x.reshape(B, dim, P)

    out = pl.pallas_call(
        functools.partial(_fused_kernel, heads=heads, head_dim=head_dim,
                          M=M, w=w, h=h, H0=H0, fold_h=fh),
        out_shape=jax.ShapeDtypeStruct((B, out_dim, P), jnp.float32),
        grid=(B // BB,),
        in_specs=[
            pl.BlockSpec(memory_space=pltpu.MemorySpace.SMEM),
            pl.BlockSpec((2 * Cd, dim), lambda g: (0, 0)),
            pl.BlockSpec((2 * Cd, 1), lambda g: (0, 0)),
            pl.BlockSpec((out_dim, Cd), lambda g: (0, 0)),
            pl.BlockSpec((out_dim, 1), lambda g: (0, 0)),
            pl.BlockSpec((P, FM), lambda g: (0, 0)),
            pl.BlockSpec((BB, dim, P), lambda g: (g, 0, 0)),
        ],
        out_specs=pl.BlockSpec((BB, out_dim, P), lambda g: (g, 0, 0)),
        compiler_params=pltpu.CompilerParams(
            dimension_semantics=("parallel",),
            vmem_limit_bytes=_VMEM_LIMIT),
    )(ab, w_cv, b_cv, wo, bo.reshape(out_dim, 1), pool, x_flat)
    return out.reshape(B, out_dim, W0, H0)


@jax.jit
def kernel(x, wp, bp, wc, bc, wo, bo, sim_alpha, sim_beta):
    return _cluster_attention(x, wp, bp, wc, bc, wo, bo, sim_alpha, sim_beta,
                              heads=4, head_dim=32, fold_w=2, fold_h=2,
                              proposal_w=2, proposal_h=2)
